# baseline (speedup 1.0000x reference)
import jax
import jax.numpy as jnp
from jax import lax
from jax.experimental import pallas as pl
from jax.experimental.pallas import tpu as pltpu

VIGILANCE = 0.75
SUB = 8
ROWS_PER_PROG = 512
LANES = 128


def _combine(s1, j1, s2, j2):
    take2 = (s2 > s1) | ((s2 == s1) & (j2 < j1))
    return jnp.maximum(s1, s2), jnp.where(take2, j2, j1)


def _body(x_ref, c_ref, out_ref, idx_ref, ctb_ref, ps_ref, pj_ref):
    d_dim = x_ref.shape[1]
    k_dim = c_ref.shape[0]
    nsub = ROWS_PER_PROG // SUB
    nchunk = k_dim // LANES

    @pl.when(pl.program_id(0) == 0)
    def _fill():
        ct = jnp.transpose(c_ref[...], (1, 0))
        for d in range(d_dim):
            ctb_ref[d] = jnp.broadcast_to(ct[d:d + 1, :], (SUB, k_dim))

    def sub_block(jb):
        base = jb * SUB
        x = x_ref[pl.ds(base, SUB), :]
        den = jnp.sum(x, axis=1, keepdims=True)
        acc = jnp.zeros((SUB, k_dim), jnp.float32)
        for d in range(d_dim):
            acc = acc + jnp.minimum(x[:, d:d + 1], ctb_ref[d])
        m = acc / den
        s = jnp.where(m >= VIGILANCE, m, jnp.zeros_like(m))
        out_ref[pl.ds(base, SUB), :] = s
        lane = lax.broadcasted_iota(jnp.int32, (SUB, LANES), 1)
        pairs = [(s[:, c * LANES:(c + 1) * LANES], lane + c * LANES)
                 for c in range(nchunk)]
        while len(pairs) > 1:
            nxt = []
            for a in range(0, len(pairs) - 1, 2):
                nxt.append(_combine(*pairs[a], *pairs[a + 1]))
            if len(pairs) % 2:
                nxt.append(pairs[-1])
            pairs = nxt
        ps_ref[jb] = pairs[0][0]
        pj_ref[jb] = pairs[0][1]

    def pair_iter(j, carry):
        sub_block(4 * j)
        sub_block(4 * j + 1)
        sub_block(4 * j + 2)
        sub_block(4 * j + 3)
        return carry

    lax.fori_loop(0, nsub // 4, pair_iter, 0)

    sv = ps_ref[...]
    jv = pj_ref[...]
    t = 1
    while t < LANES:
        sr = pltpu.roll(sv, t, 2)
        jr = pltpu.roll(jv, t, 2)
        sv, jv = _combine(sv, jv, sr, jr)
        t *= 2
    idx_ref[...] = jv[:, :, 0:1].reshape(ROWS_PER_PROG, 1)


def kernel(x, categories):
    b, d_dim = x.shape
    k_dim = categories.shape[0]
    out, idx = pl.pallas_call(
        _body,
        grid=(b // ROWS_PER_PROG,),
        in_specs=[
            pl.BlockSpec((ROWS_PER_PROG, d_dim), lambda i: (i, 0)),
            pl.BlockSpec((k_dim, d_dim), lambda i: (0, 0)),
        ],
        out_specs=[
            pl.BlockSpec((ROWS_PER_PROG, k_dim), lambda i: (i, 0)),
            pl.BlockSpec((ROWS_PER_PROG, 1), lambda i: (i, 0)),
        ],
        out_shape=[
            jax.ShapeDtypeStruct((b, k_dim), jnp.float32),
            jax.ShapeDtypeStruct((b, 1), jnp.int32),
        ],
        scratch_shapes=[
            pltpu.VMEM((d_dim, SUB, k_dim), jnp.float32),
            pltpu.VMEM((ROWS_PER_PROG // SUB, SUB, LANES), jnp.float32),
            pltpu.VMEM((ROWS_PER_PROG // SUB, SUB, LANES), jnp.int32),
        ],
    )(x, categories)
    return (out, idx.reshape(b))


# speedup vs baseline: 1.2021x; 1.2021x over previous
import jax
import jax.numpy as jnp
from jax import lax
from jax.experimental import pallas as pl
from jax.experimental.pallas import tpu as pltpu

VIGILANCE = 0.75
SUB = 8
ROWS_PER_PROG = 512
LANES = 128


def _combine(s1, j1, s2, j2):
    take2 = (s2 > s1) | ((s2 == s1) & (j2 < j1))
    return jnp.maximum(s1, s2), jnp.where(take2, j2, j1)


def _body(x_ref, c_ref, out_ref, idx_ref, ctb_ref, ps_ref, pj_ref):
    d_dim = x_ref.shape[1]
    k_dim = c_ref.shape[0]
    nsub = ROWS_PER_PROG // SUB
    nchunk = k_dim // LANES

    @pl.when(pl.program_id(0) == 0)
    def _fill():
        ct = jnp.transpose(c_ref[...], (1, 0))
        for d in range(d_dim):
            ctb_ref[d] = jnp.broadcast_to(ct[d:d + 1, :], (SUB, k_dim))

    def sub_block(jb):
        base = jb * SUB
        x = x_ref[pl.ds(base, SUB), :]
        den = jnp.sum(x, axis=1, keepdims=True)
        acc = jnp.zeros((SUB, k_dim), jnp.float32)
        for d in range(d_dim):
            acc = acc + jnp.minimum(x[:, d:d + 1], ctb_ref[d])
        m = acc / den
        s = jnp.where(m >= VIGILANCE, m, jnp.zeros_like(m))
        out_ref[pl.ds(base, SUB), :] = s
        lane = lax.broadcasted_iota(jnp.int32, (SUB, LANES), 1)
        pairs = [(s[:, c * LANES:(c + 1) * LANES], lane + c * LANES)
                 for c in range(nchunk)]
        while len(pairs) > 1:
            nxt = []
            for a in range(0, len(pairs) - 1, 2):
                nxt.append(_combine(*pairs[a], *pairs[a + 1]))
            if len(pairs) % 2:
                nxt.append(pairs[-1])
            pairs = nxt
        ps_ref[jb] = pairs[0][0]
        pj_ref[jb] = pairs[0][1]

    def pair_iter(j, carry):
        sub_block(2 * j)
        sub_block(2 * j + 1)
        return carry

    lax.fori_loop(0, nsub // 2, pair_iter, 0)

    sv = ps_ref[...]
    jv = pj_ref[...]
    t = 1
    while t < LANES:
        sr = pltpu.roll(sv, t, 2)
        jr = pltpu.roll(jv, t, 2)
        sv, jv = _combine(sv, jv, sr, jr)
        t *= 2
    idx_ref[...] = jv[:, :, 0:1].reshape(ROWS_PER_PROG, 1)


def kernel(x, categories):
    b, d_dim = x.shape
    k_dim = categories.shape[0]
    out, idx = pl.pallas_call(
        _body,
        grid=(b // ROWS_PER_PROG,),
        in_specs=[
            pl.BlockSpec((ROWS_PER_PROG, d_dim), lambda i: (i, 0)),
            pl.BlockSpec((k_dim, d_dim), lambda i: (0, 0)),
        ],
        out_specs=[
            pl.BlockSpec((ROWS_PER_PROG, k_dim), lambda i: (i, 0)),
            pl.BlockSpec((ROWS_PER_PROG, 1), lambda i: (i, 0)),
        ],
        out_shape=[
            jax.ShapeDtypeStruct((b, k_dim), jnp.float32),
            jax.ShapeDtypeStruct((b, 1), jnp.int32),
        ],
        scratch_shapes=[
            pltpu.VMEM((d_dim, SUB, k_dim), jnp.float32),
            pltpu.VMEM((ROWS_PER_PROG // SUB, SUB, LANES), jnp.float32),
            pltpu.VMEM((ROWS_PER_PROG // SUB, SUB, LANES), jnp.int32),
        ],
    )(x, categories)
    return (out, idx.reshape(b))
